# Initial kernel scaffold; baseline (speedup 1.0000x reference)
import jax
import jax.numpy as jnp
from jax.experimental import pallas as pl


def kernel(in_feat, edge_index, W_self1, W_neigh1, b1, W_self2, W_neigh2, b2):
    raise NotImplementedError("write your pallas kernel here")


# speedup vs baseline: 4.2574x; 4.2574x over previous
import functools

import jax
import jax.numpy as jnp
from jax import lax
from jax.experimental import pallas as pl
from jax.experimental.pallas import tpu as pltpu
from jax.experimental.pallas import tpu_sc as plsc

N_NODES = 10000
N_EDGES = 320000
IN_FEATS = 128
H_FEATS = 128
NUM_CLASSES = 16

NP = 10240
NC = 2
NS = 16
B = 128
CHUNKS = 80
EP = NC * NS * CHUNKS * B
ROWS_PER_TILE = NP // NS
WA = IN_FEATS + 8


def _make_sc_agg(width):
  mesh = plsc.VectorSubcoreMesh(core_axis_name="c", subcore_axis_name="s")

  @functools.partial(
      pl.kernel,
      out_type=jax.ShapeDtypeStruct((NC, NP, width), jnp.float32),
      mesh=mesh,
      compiler_params=pltpu.CompilerParams(use_tc_tiling_on_sc=False),
      scratch_types=[
          pltpu.VMEM((CHUNKS, B), jnp.int32),
          pltpu.VMEM((CHUNKS, B), jnp.int32),
          pltpu.VMEM((B, width), jnp.float32),
          pltpu.VMEM((16, width), jnp.float32),
          pltpu.VMEM_SHARED((NP, width), jnp.float32),
          pltpu.SemaphoreType.DMA,
      ],
  )
  def agg_body(src_hbm, dst_hbm, table_hbm, out_hbm, sidx, didx, rows, zrow,
               acc, sem):
    cid = lax.axis_index("c")
    sid = lax.axis_index("s")
    wid = cid * NS + sid

    pltpu.sync_copy(src_hbm.at[wid], sidx)
    pltpu.sync_copy(dst_hbm.at[wid], didx)

    zero = jnp.zeros((16,), jnp.float32)
    offs = list(range(0, width // 16 * 16, 16))
    if width % 16:
      offs.append(width - 16)
    for r in range(16):
      for off in offs:
        zrow[r, pl.ds(off, 16)] = zero
    base = sid * ROWS_PER_TILE

    def zero_body(i, _):
      pltpu.sync_copy(zrow, acc.at[pl.ds(base + i * 16, 16)])
      return 0

    lax.fori_loop(0, ROWS_PER_TILE // 16, zero_body, 0)
    plsc.subcore_barrier()

    def edge_body(j, _):
      pltpu.async_copy(table_hbm.at[sidx.at[j]], rows, sem).wait()
      pltpu.sync_copy(rows, acc.at[didx.at[j]], add=True)
      return 0

    lax.fori_loop(0, CHUNKS, edge_body, 0)
    plsc.subcore_barrier()

    pltpu.sync_copy(acc.at[pl.ds(base, ROWS_PER_TILE)],
                    out_hbm.at[cid, pl.ds(base, ROWS_PER_TILE)])

  def agg(src_r, dst_r, table):
    hbm = pltpu.MemorySpace.HBM
    return agg_body(
        pltpu.with_memory_space_constraint(src_r, hbm),
        pltpu.with_memory_space_constraint(dst_r, hbm),
        pltpu.with_memory_space_constraint(table, hbm),
    )

  return agg


_agg1 = _make_sc_agg(WA)
_agg2 = _make_sc_agg(NUM_CLASSES)

BLK = 2048


def _dense1_body(x_ref, a0_ref, a1_ref, ws1_ref, wn1_ref, b1_ref, wn2_ref,
                 h1_ref, z2_ref, rdeg_ref):
  a = a0_ref[0] + a1_ref[0]
  deg = jnp.maximum(a[:, IN_FEATS:IN_FEATS + 1], 1.0)
  rdeg = 1.0 / deg
  nbar = a[:, :IN_FEATS] * rdeg
  h1 = x_ref[...] @ ws1_ref[...] + nbar @ wn1_ref[...] + b1_ref[...]
  h1 = jnp.maximum(h1, 0.0)
  h1_ref[...] = h1
  z2_ref[...] = h1 @ wn2_ref[...]
  rdeg_ref[...] = jnp.broadcast_to(rdeg, (BLK, NUM_CLASSES))


def _dense1(x_pad, agg1, w_self1, w_neigh1, b1, w_neigh2):
  grid = (NP // BLK,)
  return pl.pallas_call(
      _dense1_body,
      grid=grid,
      in_specs=[
          pl.BlockSpec((BLK, IN_FEATS), lambda i: (i, 0)),
          pl.BlockSpec((1, BLK, WA), lambda i: (0, i, 0)),
          pl.BlockSpec((1, BLK, WA), lambda i: (1, i, 0)),
          pl.BlockSpec((IN_FEATS, H_FEATS), lambda i: (0, 0)),
          pl.BlockSpec((IN_FEATS, H_FEATS), lambda i: (0, 0)),
          pl.BlockSpec((1, H_FEATS), lambda i: (0, 0)),
          pl.BlockSpec((H_FEATS, NUM_CLASSES), lambda i: (0, 0)),
      ],
      out_specs=[
          pl.BlockSpec((BLK, H_FEATS), lambda i: (i, 0)),
          pl.BlockSpec((BLK, NUM_CLASSES), lambda i: (i, 0)),
          pl.BlockSpec((BLK, NUM_CLASSES), lambda i: (i, 0)),
      ],
      out_shape=[
          jax.ShapeDtypeStruct((NP, H_FEATS), jnp.float32),
          jax.ShapeDtypeStruct((NP, NUM_CLASSES), jnp.float32),
          jax.ShapeDtypeStruct((NP, NUM_CLASSES), jnp.float32),
      ],
  )(x_pad, agg1, agg1, w_self1, w_neigh1, b1, w_neigh2)


def _dense2_body(h1_ref, g0_ref, g1_ref, rdeg_ref, ws2_ref, b2_ref, out_ref):
  aggz = (g0_ref[0] + g1_ref[0]) * rdeg_ref[...]
  out_ref[...] = h1_ref[...] @ ws2_ref[...] + aggz + b2_ref[...]


def _dense2(h1, agg2, rdeg, w_self2, b2):
  grid = (NP // BLK,)
  return pl.pallas_call(
      _dense2_body,
      grid=grid,
      in_specs=[
          pl.BlockSpec((BLK, H_FEATS), lambda i: (i, 0)),
          pl.BlockSpec((1, BLK, NUM_CLASSES), lambda i: (0, i, 0)),
          pl.BlockSpec((1, BLK, NUM_CLASSES), lambda i: (1, i, 0)),
          pl.BlockSpec((BLK, NUM_CLASSES), lambda i: (i, 0)),
          pl.BlockSpec((H_FEATS, NUM_CLASSES), lambda i: (0, 0)),
          pl.BlockSpec((1, NUM_CLASSES), lambda i: (0, 0)),
      ],
      out_specs=pl.BlockSpec((BLK, NUM_CLASSES), lambda i: (i, 0)),
      out_shape=jax.ShapeDtypeStruct((NP, NUM_CLASSES), jnp.float32),
  )(h1, agg2, agg2, rdeg, w_self2, b2)


@jax.jit
def _run(in_feat, edge_index, w_self1, w_neigh1, b1, w_self2, w_neigh2, b2):
  src = edge_index[0].astype(jnp.int32)
  dst = edge_index[1].astype(jnp.int32)
  pad = EP - N_EDGES
  src_p = jnp.concatenate([src, jnp.zeros((pad,), jnp.int32)])
  dst_p = jnp.concatenate([dst, jnp.full((pad,), NP - 1, jnp.int32)])
  src_r = src_p.reshape(NC * NS, CHUNKS, B)
  dst_r = dst_p.reshape(NC * NS, CHUNKS, B)

  x_pad = jnp.pad(in_feat, ((0, NP - N_NODES), (0, 0)))
  xa = jnp.concatenate(
      [x_pad, jnp.ones((NP, 1), jnp.float32),
       jnp.zeros((NP, WA - IN_FEATS - 1), jnp.float32)], axis=1)

  a1 = _agg1(src_r, dst_r, xa)
  h1, z2, rdeg = _dense1(x_pad, a1, w_self1, w_neigh1,
                         b1.reshape(1, H_FEATS), w_neigh2)
  a2 = _agg2(src_r, dst_r, z2)
  out = _dense2(h1, a2, rdeg, w_self2, b2.reshape(1, NUM_CLASSES))
  return out[:N_NODES]


def kernel(in_feat, edge_index, W_self1, W_neigh1, b1, W_self2, W_neigh2, b2):
  return _run(in_feat, edge_index, W_self1, W_neigh1, b1, W_self2, W_neigh2,
              b2)


# speedup vs baseline: 5.6837x; 1.3350x over previous
import functools

import jax
import jax.numpy as jnp
from jax import lax
from jax.experimental import pallas as pl
from jax.experimental.pallas import tpu as pltpu
from jax.experimental.pallas import tpu_sc as plsc

N_NODES = 10000
N_EDGES = 320000
IN_FEATS = 128
H_FEATS = 128
NUM_CLASSES = 16

NP = 10240
NC = 2
NS = 16
NT = NC * NS
ROWS_PER_TILE = NP // NS
NSLOT = 4

B1, CHUNKS1 = 48, 212
B2, CHUNKS2 = 128, 80
EP1 = NT * CHUNKS1 * B1
EP2 = NT * CHUNKS2 * B2
WD = 16


def _make_sc_agg(width, bsz, chunks):
  mesh = plsc.VectorSubcoreMesh(core_axis_name="c", subcore_axis_name="s")

  @functools.partial(
      pl.kernel,
      out_type=jax.ShapeDtypeStruct((NC, NP, width), jnp.float32),
      mesh=mesh,
      compiler_params=pltpu.CompilerParams(use_tc_tiling_on_sc=False),
      scratch_types=[
          pltpu.VMEM((chunks, bsz), jnp.int32),
          pltpu.VMEM((chunks, bsz), jnp.int32),
          [pltpu.VMEM((bsz, width), jnp.float32) for _ in range(NSLOT)],
          pltpu.VMEM_SHARED((NP, width), jnp.float32),
          [pltpu.SemaphoreType.DMA for _ in range(NSLOT)],
          [pltpu.SemaphoreType.DMA for _ in range(NSLOT)],
      ],
  )
  def agg_body(src_hbm, dst_hbm, table_hbm, out_hbm, sidx, didx, rows, acc,
               gsem, ssem):
    cid = lax.axis_index("c")
    sid = lax.axis_index("s")
    wid = cid * NS + sid

    pltpu.sync_copy(src_hbm.at[wid], sidx)
    pltpu.sync_copy(dst_hbm.at[wid], didx)

    zero = jnp.zeros((16,), jnp.float32)
    for r in range(16):
      for c in range(width // 16):
        rows[0][r, pl.ds(c * 16, 16)] = zero
    base = sid * ROWS_PER_TILE

    def zero_body(i, _):
      pltpu.sync_copy(rows[0].at[pl.ds(0, 16)],
                      acc.at[pl.ds(base + i * 16, 16)])
      return 0

    lax.fori_loop(0, ROWS_PER_TILE // 16, zero_body, 0)
    plsc.subcore_barrier()

    def gather_start(j, k):
      pltpu.async_copy(table_hbm.at[sidx.at[j]], rows[k], gsem[k])

    def gather_wait(j, k):
      pltpu.make_async_copy(table_hbm.at[sidx.at[j]], rows[k], gsem[k]).wait()

    def scatter_start(j, k):
      pltpu.async_copy(rows[k], acc.at[didx.at[j]], ssem[k], add=True)

    def scatter_wait(j, k):
      pltpu.make_async_copy(rows[k], acc.at[didx.at[j]], ssem[k]).wait()

    nr = chunks // NSLOT
    for k in range(NSLOT):
      gather_start(k, k)

    def edge_round(g, _):
      for k in range(NSLOT):
        j = NSLOT * g + k
        gather_wait(j, k)
        scatter_start(j, k)
      for k in range(NSLOT):
        j = NSLOT * g + k
        scatter_wait(j, k)
        gather_start(j + NSLOT, k)
      return 0

    lax.fori_loop(0, nr - 1, edge_round, 0)
    for k in range(NSLOT):
      j = NSLOT * (nr - 1) + k
      gather_wait(j, k)
      scatter_start(j, k)
    for k in range(NSLOT):
      j = NSLOT * (nr - 1) + k
      scatter_wait(j, k)
    plsc.subcore_barrier()

    pltpu.sync_copy(acc.at[pl.ds(base, ROWS_PER_TILE)],
                    out_hbm.at[cid, pl.ds(base, ROWS_PER_TILE)])

  return agg_body


def _make_sc_deg():
  mesh = plsc.VectorSubcoreMesh(core_axis_name="c", subcore_axis_name="s")

  @functools.partial(
      pl.kernel,
      out_type=jax.ShapeDtypeStruct((NC, NP, WD), jnp.float32),
      mesh=mesh,
      compiler_params=pltpu.CompilerParams(use_tc_tiling_on_sc=False),
      scratch_types=[
          pltpu.VMEM((CHUNKS2, B2), jnp.int32),
          pltpu.VMEM((B2, WD), jnp.float32),
          pltpu.VMEM((16, WD), jnp.float32),
          pltpu.VMEM_SHARED((NP, WD), jnp.float32),
          [pltpu.SemaphoreType.DMA for _ in range(NSLOT)],
      ],
  )
  def deg_body(dst_hbm, out_hbm, didx, ones, zb, acc, ssem):
    cid = lax.axis_index("c")
    sid = lax.axis_index("s")
    wid = cid * NS + sid

    pltpu.sync_copy(dst_hbm.at[wid], didx)

    one = jnp.full((16,), 1.0, jnp.float32)
    zero = jnp.zeros((16,), jnp.float32)
    for r in range(B2):
      ones[r, pl.ds(0, WD)] = one
    for r in range(16):
      zb[r, pl.ds(0, WD)] = zero
    base = sid * ROWS_PER_TILE

    def zero_body(i, _):
      pltpu.sync_copy(zb, acc.at[pl.ds(base + i * 16, 16)])
      return 0

    lax.fori_loop(0, ROWS_PER_TILE // 16, zero_body, 0)
    plsc.subcore_barrier()

    def sstart(j, k):
      pltpu.async_copy(ones, acc.at[didx.at[j]], ssem[k], add=True)

    def swait(j, k):
      pltpu.make_async_copy(ones, acc.at[didx.at[j]], ssem[k]).wait()

    nr = CHUNKS2 // NSLOT
    for k in range(NSLOT):
      sstart(k, k)

    def rnd(g, _):
      for k in range(NSLOT):
        swait(NSLOT * (g - 1) + k, k)
        sstart(NSLOT * g + k, k)
      return 0

    lax.fori_loop(1, nr, rnd, 0)
    for k in range(NSLOT):
      swait(NSLOT * (nr - 1) + k, k)
    plsc.subcore_barrier()

    pltpu.sync_copy(acc.at[pl.ds(base, ROWS_PER_TILE)],
                    out_hbm.at[cid, pl.ds(base, ROWS_PER_TILE)])

  return deg_body


_agg1 = _make_sc_agg(IN_FEATS, B1, CHUNKS1)
_agg2 = _make_sc_agg(NUM_CLASSES, B2, CHUNKS2)
_deg = _make_sc_deg()

BLK = 2048


def _dense1_body(x_ref, a0_ref, a1_ref, d0_ref, d1_ref, ws1_ref, wn1_ref,
                 b1_ref, wn2_ref, h1_ref, z2_ref, rdeg_ref):
  a = a0_ref[0] + a1_ref[0]
  deg = jnp.maximum(d0_ref[0][:, 0:1] + d1_ref[0][:, 0:1], 1.0)
  rdeg = 1.0 / deg
  nbar = a * rdeg
  h1 = x_ref[...] @ ws1_ref[...] + nbar @ wn1_ref[...] + b1_ref[...]
  h1 = jnp.maximum(h1, 0.0)
  h1_ref[...] = h1
  z2_ref[...] = h1 @ wn2_ref[...]
  rdeg_ref[...] = jnp.broadcast_to(rdeg, (BLK, NUM_CLASSES))


def _dense1(x_pad, agg1, deg, w_self1, w_neigh1, b1, w_neigh2):
  grid = (NP // BLK,)
  return pl.pallas_call(
      _dense1_body,
      grid=grid,
      in_specs=[
          pl.BlockSpec((BLK, IN_FEATS), lambda i: (i, 0)),
          pl.BlockSpec((1, BLK, IN_FEATS), lambda i: (0, i, 0)),
          pl.BlockSpec((1, BLK, IN_FEATS), lambda i: (1, i, 0)),
          pl.BlockSpec((1, BLK, WD), lambda i: (0, i, 0)),
          pl.BlockSpec((1, BLK, WD), lambda i: (1, i, 0)),
          pl.BlockSpec((IN_FEATS, H_FEATS), lambda i: (0, 0)),
          pl.BlockSpec((IN_FEATS, H_FEATS), lambda i: (0, 0)),
          pl.BlockSpec((1, H_FEATS), lambda i: (0, 0)),
          pl.BlockSpec((H_FEATS, NUM_CLASSES), lambda i: (0, 0)),
      ],
      out_specs=[
          pl.BlockSpec((BLK, H_FEATS), lambda i: (i, 0)),
          pl.BlockSpec((BLK, NUM_CLASSES), lambda i: (i, 0)),
          pl.BlockSpec((BLK, NUM_CLASSES), lambda i: (i, 0)),
      ],
      out_shape=[
          jax.ShapeDtypeStruct((NP, H_FEATS), jnp.float32),
          jax.ShapeDtypeStruct((NP, NUM_CLASSES), jnp.float32),
          jax.ShapeDtypeStruct((NP, NUM_CLASSES), jnp.float32),
      ],
  )(x_pad, agg1, agg1, deg, deg, w_self1, w_neigh1, b1, w_neigh2)


def _dense2_body(h1_ref, g0_ref, g1_ref, rdeg_ref, ws2_ref, b2_ref, out_ref):
  aggz = (g0_ref[0] + g1_ref[0]) * rdeg_ref[...]
  out_ref[...] = h1_ref[...] @ ws2_ref[...] + aggz + b2_ref[...]


def _dense2(h1, agg2, rdeg, w_self2, b2):
  grid = (NP // BLK,)
  return pl.pallas_call(
      _dense2_body,
      grid=grid,
      in_specs=[
          pl.BlockSpec((BLK, H_FEATS), lambda i: (i, 0)),
          pl.BlockSpec((1, BLK, NUM_CLASSES), lambda i: (0, i, 0)),
          pl.BlockSpec((1, BLK, NUM_CLASSES), lambda i: (1, i, 0)),
          pl.BlockSpec((BLK, NUM_CLASSES), lambda i: (i, 0)),
          pl.BlockSpec((H_FEATS, NUM_CLASSES), lambda i: (0, 0)),
          pl.BlockSpec((1, NUM_CLASSES), lambda i: (0, 0)),
      ],
      out_specs=pl.BlockSpec((BLK, NUM_CLASSES), lambda i: (i, 0)),
      out_shape=jax.ShapeDtypeStruct((NP, NUM_CLASSES), jnp.float32),
  )(h1, agg2, agg2, rdeg, w_self2, b2)


def _pad_edges(src, dst, ep, nchunks, bsz):
  pad = ep - N_EDGES
  src_p = jnp.concatenate([src, jnp.zeros((pad,), jnp.int32)])
  dst_p = jnp.concatenate([dst, jnp.full((pad,), NP - 1, jnp.int32)])
  return src_p.reshape(NT, nchunks, bsz), dst_p.reshape(NT, nchunks, bsz)


@jax.jit
def _run(in_feat, edge_index, w_self1, w_neigh1, b1, w_self2, w_neigh2, b2):
  src = edge_index[0].astype(jnp.int32)
  dst = edge_index[1].astype(jnp.int32)
  src1, dst1 = _pad_edges(src, dst, EP1, CHUNKS1, B1)
  src2, dst2 = _pad_edges(src, dst, EP2, CHUNKS2, B2)
  x_pad = jnp.pad(in_feat, ((0, NP - N_NODES), (0, 0)))

  deg = _deg(dst2)
  a1 = _agg1(src1, dst1, x_pad)
  h1, z2, rdeg = _dense1(x_pad, a1, deg, w_self1, w_neigh1,
                         b1.reshape(1, H_FEATS), w_neigh2)
  a2 = _agg2(src2, dst2, z2)
  out = _dense2(h1, a2, rdeg, w_self2, b2.reshape(1, NUM_CLASSES))
  return out[:N_NODES]


def kernel(in_feat, edge_index, W_self1, W_neigh1, b1, W_self2, W_neigh2, b2):
  return _run(in_feat, edge_index, W_self1, W_neigh1, b1, W_self2, W_neigh2,
              b2)


# speedup vs baseline: 9.5404x; 1.6786x over previous
import functools

import jax
import jax.numpy as jnp
from jax import lax
from jax.experimental import pallas as pl
from jax.experimental.pallas import tpu as pltpu
from jax.experimental.pallas import tpu_sc as plsc

N_NODES = 10000
N_EDGES = 320000
IN_FEATS = 128
H_FEATS = 128
NUM_CLASSES = 16

NP = 10240
NC = 2
NS = 16
NT = NC * NS
ROWS_PER_TILE = NP // NS
NSLOT = 4

B1, CHUNKS1 = 64, 316
B2, CHUNKS2 = 128, 80
EP1 = NS * CHUNKS1 * B1
EP2 = NT * CHUNKS2 * B2
WD = 16
HW = IN_FEATS // 2


def _make_sc_agg1():
  mesh = plsc.VectorSubcoreMesh(core_axis_name="c", subcore_axis_name="s")

  @functools.partial(
      pl.kernel,
      out_type=jax.ShapeDtypeStruct((NC, NP, HW), jnp.float32),
      mesh=mesh,
      compiler_params=pltpu.CompilerParams(use_tc_tiling_on_sc=False),
      scratch_types=[
          pltpu.VMEM((CHUNKS1, B1), jnp.int32),
          [pltpu.VMEM((B1,), jnp.int32) for _ in range(NSLOT)],
          [pltpu.VMEM((B1,), jnp.int32) for _ in range(NSLOT)],
          [pltpu.VMEM((B1, HW), jnp.float32) for _ in range(NSLOT)],
          pltpu.VMEM_SHARED((NP, HW), jnp.float32),
          pltpu.VMEM_SHARED((NP, HW), jnp.float32),
          [pltpu.SemaphoreType.DMA for _ in range(NSLOT)],
          [pltpu.SemaphoreType.DMA for _ in range(NSLOT)],
      ],
  )
  def agg1_body(pk_hbm, x_hbm, out_hbm, pidx, sidx, didx, rows, table,
                acc, gsem, ssem):
    cid = lax.axis_index("c")
    sid = lax.axis_index("s")
    base = sid * ROWS_PER_TILE

    pltpu.sync_copy(pk_hbm.at[sid], pidx)
    pltpu.sync_copy(x_hbm.at[cid, pl.ds(base, ROWS_PER_TILE)],
                    table.at[pl.ds(base, ROWS_PER_TILE)])

    def unpack(j, k):
      for c in range(B1 // 16):
        v = pidx[j, pl.ds(c * 16, 16)]
        sidx[k][pl.ds(c * 16, 16)] = lax.shift_right_logical(v, 14)
        didx[k][pl.ds(c * 16, 16)] = lax.bitwise_and(v, 16383)

    zero = jnp.zeros((16,), jnp.float32)
    for r in range(16):
      for c in range(HW // 16):
        rows[0][r, pl.ds(c * 16, 16)] = zero

    def zero_body(i, _):
      pltpu.sync_copy(rows[0].at[pl.ds(0, 16)],
                      acc.at[pl.ds(base + i * 16, 16)])
      return 0

    lax.fori_loop(0, ROWS_PER_TILE // 16, zero_body, 0)
    plsc.subcore_barrier()

    def gather_start(k):
      pltpu.async_copy(table.at[sidx[k]], rows[k], gsem[k])

    def gather_wait(k):
      pltpu.make_async_copy(table.at[sidx[k]], rows[k], gsem[k]).wait()

    def scatter_start(k):
      pltpu.async_copy(rows[k], acc.at[didx[k]], ssem[k], add=True)

    def scatter_wait(k):
      pltpu.make_async_copy(rows[k], acc.at[didx[k]], ssem[k]).wait()

    nr = CHUNKS1 // NSLOT
    for k in range(NSLOT):
      unpack(k, k)
      gather_start(k)

    def edge_round(g, _):
      for k in range(NSLOT):
        gather_wait(k)
        scatter_start(k)
      for k in range(NSLOT):
        scatter_wait(k)
        unpack(NSLOT * g + k + NSLOT, k)
        gather_start(k)
      return 0

    lax.fori_loop(0, nr - 1, edge_round, 0)
    for k in range(NSLOT):
      gather_wait(k)
      scatter_start(k)
    for k in range(NSLOT):
      scatter_wait(k)
    plsc.subcore_barrier()

    pltpu.sync_copy(acc.at[pl.ds(base, ROWS_PER_TILE)],
                    out_hbm.at[cid, pl.ds(base, ROWS_PER_TILE)])

  return agg1_body


def _make_sc_agg(width, bsz, chunks):
  mesh = plsc.VectorSubcoreMesh(core_axis_name="c", subcore_axis_name="s")

  @functools.partial(
      pl.kernel,
      out_type=jax.ShapeDtypeStruct((NC, NP, width), jnp.float32),
      mesh=mesh,
      compiler_params=pltpu.CompilerParams(use_tc_tiling_on_sc=False),
      scratch_types=[
          pltpu.VMEM((chunks, bsz), jnp.int32),
          pltpu.VMEM((chunks, bsz), jnp.int32),
          [pltpu.VMEM((bsz, width), jnp.float32) for _ in range(NSLOT)],
          pltpu.VMEM_SHARED((NP, width), jnp.float32),
          [pltpu.SemaphoreType.DMA for _ in range(NSLOT)],
          [pltpu.SemaphoreType.DMA for _ in range(NSLOT)],
      ],
  )
  def agg_body(src_hbm, dst_hbm, table_hbm, out_hbm, sidx, didx, rows, acc,
               gsem, ssem):
    cid = lax.axis_index("c")
    sid = lax.axis_index("s")
    wid = cid * NS + sid

    pltpu.sync_copy(src_hbm.at[wid], sidx)
    pltpu.sync_copy(dst_hbm.at[wid], didx)

    zero = jnp.zeros((16,), jnp.float32)
    for r in range(16):
      for c in range(width // 16):
        rows[0][r, pl.ds(c * 16, 16)] = zero
    base = sid * ROWS_PER_TILE

    def zero_body(i, _):
      pltpu.sync_copy(rows[0].at[pl.ds(0, 16)],
                      acc.at[pl.ds(base + i * 16, 16)])
      return 0

    lax.fori_loop(0, ROWS_PER_TILE // 16, zero_body, 0)
    plsc.subcore_barrier()

    def gather_start(j, k):
      pltpu.async_copy(table_hbm.at[sidx.at[j]], rows[k], gsem[k])

    def gather_wait(j, k):
      pltpu.make_async_copy(table_hbm.at[sidx.at[j]], rows[k], gsem[k]).wait()

    def scatter_start(j, k):
      pltpu.async_copy(rows[k], acc.at[didx.at[j]], ssem[k], add=True)

    def scatter_wait(j, k):
      pltpu.make_async_copy(rows[k], acc.at[didx.at[j]], ssem[k]).wait()

    nr = chunks // NSLOT
    for k in range(NSLOT):
      gather_start(k, k)

    def edge_round(g, _):
      for k in range(NSLOT):
        j = NSLOT * g + k
        gather_wait(j, k)
        scatter_start(j, k)
      for k in range(NSLOT):
        j = NSLOT * g + k
        scatter_wait(j, k)
        gather_start(j + NSLOT, k)
      return 0

    lax.fori_loop(0, nr - 1, edge_round, 0)
    for k in range(NSLOT):
      j = NSLOT * (nr - 1) + k
      gather_wait(j, k)
      scatter_start(j, k)
    for k in range(NSLOT):
      j = NSLOT * (nr - 1) + k
      scatter_wait(j, k)
    plsc.subcore_barrier()

    pltpu.sync_copy(acc.at[pl.ds(base, ROWS_PER_TILE)],
                    out_hbm.at[cid, pl.ds(base, ROWS_PER_TILE)])

  return agg_body


def _make_sc_deg():
  mesh = plsc.VectorSubcoreMesh(core_axis_name="c", subcore_axis_name="s")

  @functools.partial(
      pl.kernel,
      out_type=jax.ShapeDtypeStruct((NC, NP, WD), jnp.float32),
      mesh=mesh,
      compiler_params=pltpu.CompilerParams(use_tc_tiling_on_sc=False),
      scratch_types=[
          pltpu.VMEM((CHUNKS2, B2), jnp.int32),
          pltpu.VMEM((B2, WD), jnp.float32),
          pltpu.VMEM((16, WD), jnp.float32),
          pltpu.VMEM_SHARED((NP, WD), jnp.float32),
          [pltpu.SemaphoreType.DMA for _ in range(NSLOT)],
      ],
  )
  def deg_body(dst_hbm, out_hbm, didx, ones, zb, acc, ssem):
    cid = lax.axis_index("c")
    sid = lax.axis_index("s")
    wid = cid * NS + sid

    pltpu.sync_copy(dst_hbm.at[wid], didx)

    one = jnp.full((16,), 1.0, jnp.float32)
    zero = jnp.zeros((16,), jnp.float32)
    for r in range(B2):
      ones[r, pl.ds(0, WD)] = one
    for r in range(16):
      zb[r, pl.ds(0, WD)] = zero
    base = sid * ROWS_PER_TILE

    def zero_body(i, _):
      pltpu.sync_copy(zb, acc.at[pl.ds(base + i * 16, 16)])
      return 0

    lax.fori_loop(0, ROWS_PER_TILE // 16, zero_body, 0)
    plsc.subcore_barrier()

    def sstart(j, k):
      pltpu.async_copy(ones, acc.at[didx.at[j]], ssem[k], add=True)

    def swait(j, k):
      pltpu.make_async_copy(ones, acc.at[didx.at[j]], ssem[k]).wait()

    nr = CHUNKS2 // NSLOT
    for k in range(NSLOT):
      sstart(k, k)

    def rnd(g, _):
      for k in range(NSLOT):
        swait(NSLOT * (g - 1) + k, k)
        sstart(NSLOT * g + k, k)
      return 0

    lax.fori_loop(1, nr, rnd, 0)
    for k in range(NSLOT):
      swait(NSLOT * (nr - 1) + k, k)
    plsc.subcore_barrier()

    pltpu.sync_copy(acc.at[pl.ds(base, ROWS_PER_TILE)],
                    out_hbm.at[cid, pl.ds(base, ROWS_PER_TILE)])

  return deg_body


_agg1 = _make_sc_agg1()
_agg2 = _make_sc_agg(NUM_CLASSES, B2, CHUNKS2)
_deg = _make_sc_deg()

BLK = 2048


def _dense1_body(x_ref, a0_ref, a1_ref, d0_ref, d1_ref, ws1_ref, wn1_ref,
                 b1_ref, wn2_ref, h1_ref, z2_ref, rdeg_ref):
  a = jnp.concatenate([a0_ref[0], a1_ref[0]], axis=1)
  deg = jnp.maximum(d0_ref[0][:, 0:1] + d1_ref[0][:, 0:1], 1.0)
  rdeg = 1.0 / deg
  nbar = a * rdeg
  h1 = x_ref[...] @ ws1_ref[...] + nbar @ wn1_ref[...] + b1_ref[...]
  h1 = jnp.maximum(h1, 0.0)
  h1_ref[...] = h1
  z2_ref[...] = h1 @ wn2_ref[...]
  rdeg_ref[...] = jnp.broadcast_to(rdeg, (BLK, NUM_CLASSES))


def _dense1(x_pad, agg1, deg, w_self1, w_neigh1, b1, w_neigh2):
  grid = (NP // BLK,)
  return pl.pallas_call(
      _dense1_body,
      grid=grid,
      in_specs=[
          pl.BlockSpec((BLK, IN_FEATS), lambda i: (i, 0)),
          pl.BlockSpec((1, BLK, HW), lambda i: (0, i, 0)),
          pl.BlockSpec((1, BLK, HW), lambda i: (1, i, 0)),
          pl.BlockSpec((1, BLK, WD), lambda i: (0, i, 0)),
          pl.BlockSpec((1, BLK, WD), lambda i: (1, i, 0)),
          pl.BlockSpec((IN_FEATS, H_FEATS), lambda i: (0, 0)),
          pl.BlockSpec((IN_FEATS, H_FEATS), lambda i: (0, 0)),
          pl.BlockSpec((1, H_FEATS), lambda i: (0, 0)),
          pl.BlockSpec((H_FEATS, NUM_CLASSES), lambda i: (0, 0)),
      ],
      out_specs=[
          pl.BlockSpec((BLK, H_FEATS), lambda i: (i, 0)),
          pl.BlockSpec((BLK, NUM_CLASSES), lambda i: (i, 0)),
          pl.BlockSpec((BLK, NUM_CLASSES), lambda i: (i, 0)),
      ],
      out_shape=[
          jax.ShapeDtypeStruct((NP, H_FEATS), jnp.float32),
          jax.ShapeDtypeStruct((NP, NUM_CLASSES), jnp.float32),
          jax.ShapeDtypeStruct((NP, NUM_CLASSES), jnp.float32),
      ],
  )(x_pad, agg1, agg1, deg, deg, w_self1, w_neigh1, b1, w_neigh2)


def _dense2_body(h1_ref, g0_ref, g1_ref, rdeg_ref, ws2_ref, b2_ref, out_ref):
  aggz = (g0_ref[0] + g1_ref[0]) * rdeg_ref[...]
  out_ref[...] = h1_ref[...] @ ws2_ref[...] + aggz + b2_ref[...]


def _dense2(h1, agg2, rdeg, w_self2, b2):
  grid = (NP // BLK,)
  return pl.pallas_call(
      _dense2_body,
      grid=grid,
      in_specs=[
          pl.BlockSpec((BLK, H_FEATS), lambda i: (i, 0)),
          pl.BlockSpec((1, BLK, NUM_CLASSES), lambda i: (0, i, 0)),
          pl.BlockSpec((1, BLK, NUM_CLASSES), lambda i: (1, i, 0)),
          pl.BlockSpec((BLK, NUM_CLASSES), lambda i: (i, 0)),
          pl.BlockSpec((H_FEATS, NUM_CLASSES), lambda i: (0, 0)),
          pl.BlockSpec((1, NUM_CLASSES), lambda i: (0, 0)),
      ],
      out_specs=pl.BlockSpec((BLK, NUM_CLASSES), lambda i: (i, 0)),
      out_shape=jax.ShapeDtypeStruct((NP, NUM_CLASSES), jnp.float32),
  )(h1, agg2, agg2, rdeg, w_self2, b2)


def _pad_edges(src, dst, ep, nsplit, nchunks, bsz):
  pad = ep - N_EDGES
  src_p = jnp.concatenate([src, jnp.zeros((pad,), jnp.int32)])
  dst_p = jnp.concatenate([dst, jnp.full((pad,), NP - 1, jnp.int32)])
  return src_p.reshape(nsplit, nchunks, bsz), dst_p.reshape(nsplit, nchunks, bsz)


@jax.jit
def _run(in_feat, edge_index, w_self1, w_neigh1, b1, w_self2, w_neigh2, b2):
  src = edge_index[0].astype(jnp.int32)
  dst = edge_index[1].astype(jnp.int32)
  pk = src * 16384 + dst
  pk1 = jnp.concatenate(
      [pk, jnp.full((EP1 - N_EDGES,), NP - 1, jnp.int32)]
  ).reshape(NS, CHUNKS1, B1)
  src2, dst2 = _pad_edges(src, dst, EP2, NT, CHUNKS2, B2)
  x_pad = jnp.pad(in_feat, ((0, NP - N_NODES), (0, 0)))
  x_sc = x_pad.reshape(NP, NC, HW).transpose(1, 0, 2)

  deg = _deg(dst2)
  a1 = _agg1(pk1, x_sc)
  h1, z2, rdeg = _dense1(x_pad, a1, deg, w_self1, w_neigh1,
                         b1.reshape(1, H_FEATS), w_neigh2)
  a2 = _agg2(src2, dst2, z2)
  out = _dense2(h1, a2, rdeg, w_self2, b2.reshape(1, NUM_CLASSES))
  return out[:N_NODES]


def kernel(in_feat, edge_index, W_self1, W_neigh1, b1, W_self2, W_neigh2, b2):
  return _run(in_feat, edge_index, W_self1, W_neigh1, b1, W_self2, W_neigh2,
              b2)


# speedup vs baseline: 10.1498x; 1.0639x over previous
import functools

import jax
import jax.numpy as jnp
from jax import lax
from jax.experimental import pallas as pl
from jax.experimental.pallas import tpu as pltpu
from jax.experimental.pallas import tpu_sc as plsc

N_NODES = 10000
N_EDGES = 320000
IN_FEATS = 128
H_FEATS = 128
NUM_CLASSES = 16

NP = 10240
NC = 2
NS = 16
NT = NC * NS
ROWS_PER_TILE = NP // NS
NSLOT = 4

B1, CHUNKS1 = 64, 316
B2, CHUNKS2 = 128, 80
EP1 = NS * CHUNKS1 * B1
EP2 = NT * CHUNKS2 * B2
WD = 16
HW = IN_FEATS // 2


def _make_sc_agg1():
  mesh = plsc.VectorSubcoreMesh(core_axis_name="c", subcore_axis_name="s")

  @functools.partial(
      pl.kernel,
      out_type=[
          jax.ShapeDtypeStruct((NC, NP, HW), jnp.float32),
          jax.ShapeDtypeStruct((NC, NP, WD), jnp.float32),
      ],
      mesh=mesh,
      compiler_params=pltpu.CompilerParams(use_tc_tiling_on_sc=False),
      scratch_types=[
          pltpu.VMEM((CHUNKS1, B1), jnp.int32),
          [pltpu.VMEM((B1,), jnp.int32) for _ in range(NSLOT)],
          [pltpu.VMEM((B1,), jnp.int32) for _ in range(NSLOT)],
          [pltpu.VMEM((B1, HW), jnp.float32) for _ in range(NSLOT)],
          pltpu.VMEM((B1, WD), jnp.float32),
          pltpu.VMEM_SHARED((NP, HW), jnp.float32),
          pltpu.VMEM_SHARED((NP, HW), jnp.float32),
          pltpu.VMEM_SHARED((NP, WD), jnp.float32),
          [pltpu.SemaphoreType.DMA for _ in range(NSLOT)],
          [pltpu.SemaphoreType.DMA for _ in range(NSLOT)],
          [pltpu.SemaphoreType.DMA for _ in range(NSLOT)],
      ],
  )
  def agg1_body(pk_hbm, x_hbm, out_hbm, deg_hbm, pidx, sidx, didx, rows,
                ones, table, acc, dacc, gsem, ssem, dsem):
    cid = lax.axis_index("c")
    sid = lax.axis_index("s")
    base = sid * ROWS_PER_TILE

    pltpu.sync_copy(pk_hbm.at[sid], pidx)
    pltpu.sync_copy(x_hbm.at[cid, pl.ds(base, ROWS_PER_TILE)],
                    table.at[pl.ds(base, ROWS_PER_TILE)])

    def unpack(j, k):
      for c in range(B1 // 16):
        v = pidx[j, pl.ds(c * 16, 16)]
        sidx[k][pl.ds(c * 16, 16)] = lax.shift_right_logical(v, 14)
        didx[k][pl.ds(c * 16, 16)] = lax.bitwise_and(v, 16383)

    zero = jnp.zeros((16,), jnp.float32)
    one = jnp.full((16,), 1.0, jnp.float32)
    for r in range(16):
      for c in range(HW // 16):
        rows[0][r, pl.ds(c * 16, 16)] = zero
      ones[r, pl.ds(0, WD)] = zero

    def zero_body(i, _):
      pltpu.sync_copy(rows[0].at[pl.ds(0, 16)],
                      acc.at[pl.ds(base + i * 16, 16)])
      pltpu.sync_copy(ones.at[pl.ds(0, 16)],
                      dacc.at[pl.ds(base + i * 16, 16)])
      return 0

    lax.fori_loop(0, ROWS_PER_TILE // 16, zero_body, 0)
    for r in range(B1):
      ones[r, pl.ds(0, WD)] = one
    plsc.subcore_barrier()

    def gather_start(k):
      pltpu.async_copy(table.at[sidx[k]], rows[k], gsem[k])

    def gather_wait(k):
      pltpu.make_async_copy(table.at[sidx[k]], rows[k], gsem[k]).wait()

    def scatter_start(k):
      pltpu.async_copy(rows[k], acc.at[didx[k]], ssem[k], add=True)
      pltpu.async_copy(ones, dacc.at[didx[k]], dsem[k], add=True)

    def scatter_wait(k):
      pltpu.make_async_copy(rows[k], acc.at[didx[k]], ssem[k]).wait()
      pltpu.make_async_copy(ones, dacc.at[didx[k]], dsem[k]).wait()

    nr = CHUNKS1 // NSLOT
    for k in range(NSLOT):
      unpack(k, k)
      gather_start(k)

    def edge_round(g, _):
      for k in range(NSLOT):
        gather_wait(k)
        scatter_start(k)
      for k in range(NSLOT):
        scatter_wait(k)
        unpack(NSLOT * g + k + NSLOT, k)
        gather_start(k)
      return 0

    lax.fori_loop(0, nr - 1, edge_round, 0)
    for k in range(NSLOT):
      gather_wait(k)
      scatter_start(k)
    for k in range(NSLOT):
      scatter_wait(k)
    plsc.subcore_barrier()

    pltpu.sync_copy(acc.at[pl.ds(base, ROWS_PER_TILE)],
                    out_hbm.at[cid, pl.ds(base, ROWS_PER_TILE)])
    pltpu.sync_copy(dacc.at[pl.ds(base, ROWS_PER_TILE)],
                    deg_hbm.at[cid, pl.ds(base, ROWS_PER_TILE)])

  return agg1_body


def _make_sc_agg2(width, bsz, chunks):
  mesh = plsc.VectorSubcoreMesh(core_axis_name="c", subcore_axis_name="s")

  @functools.partial(
      pl.kernel,
      out_type=jax.ShapeDtypeStruct((NC, NP, width), jnp.float32),
      mesh=mesh,
      compiler_params=pltpu.CompilerParams(use_tc_tiling_on_sc=False),
      scratch_types=[
          pltpu.VMEM((chunks, bsz), jnp.int32),
          pltpu.VMEM((chunks, bsz), jnp.int32),
          [pltpu.VMEM((bsz, width), jnp.float32) for _ in range(NSLOT)],
          pltpu.VMEM_SHARED((NP, width), jnp.float32),
          pltpu.VMEM_SHARED((NP, width), jnp.float32),
          [pltpu.SemaphoreType.DMA for _ in range(NSLOT)],
          [pltpu.SemaphoreType.DMA for _ in range(NSLOT)],
      ],
  )
  def agg_body(src_hbm, dst_hbm, z_hbm, out_hbm, sidx, didx, rows, table,
               acc, gsem, ssem):
    cid = lax.axis_index("c")
    sid = lax.axis_index("s")
    wid = cid * NS + sid
    base = sid * ROWS_PER_TILE

    pltpu.sync_copy(src_hbm.at[wid], sidx)
    pltpu.sync_copy(dst_hbm.at[wid], didx)
    pltpu.sync_copy(z_hbm.at[pl.ds(base, ROWS_PER_TILE)],
                    table.at[pl.ds(base, ROWS_PER_TILE)])

    zero = jnp.zeros((16,), jnp.float32)
    for r in range(16):
      for c in range(width // 16):
        rows[0][r, pl.ds(c * 16, 16)] = zero

    def zero_body(i, _):
      pltpu.sync_copy(rows[0].at[pl.ds(0, 16)],
                      acc.at[pl.ds(base + i * 16, 16)])
      return 0

    lax.fori_loop(0, ROWS_PER_TILE // 16, zero_body, 0)
    plsc.subcore_barrier()

    def gather_start(j, k):
      pltpu.async_copy(table.at[sidx.at[j]], rows[k], gsem[k])

    def gather_wait(j, k):
      pltpu.make_async_copy(table.at[sidx.at[j]], rows[k], gsem[k]).wait()

    def scatter_start(j, k):
      pltpu.async_copy(rows[k], acc.at[didx.at[j]], ssem[k], add=True)

    def scatter_wait(j, k):
      pltpu.make_async_copy(rows[k], acc.at[didx.at[j]], ssem[k]).wait()

    nr = chunks // NSLOT
    for k in range(NSLOT):
      gather_start(k, k)

    def edge_round(g, _):
      for k in range(NSLOT):
        j = NSLOT * g + k
        gather_wait(j, k)
        scatter_start(j, k)
      for k in range(NSLOT):
        j = NSLOT * g + k
        scatter_wait(j, k)
        gather_start(j + NSLOT, k)
      return 0

    lax.fori_loop(0, nr - 1, edge_round, 0)
    for k in range(NSLOT):
      j = NSLOT * (nr - 1) + k
      gather_wait(j, k)
      scatter_start(j, k)
    for k in range(NSLOT):
      j = NSLOT * (nr - 1) + k
      scatter_wait(j, k)
    plsc.subcore_barrier()

    pltpu.sync_copy(acc.at[pl.ds(base, ROWS_PER_TILE)],
                    out_hbm.at[cid, pl.ds(base, ROWS_PER_TILE)])

  return agg_body


_agg1 = _make_sc_agg1()
_agg2 = _make_sc_agg2(NUM_CLASSES, B2, CHUNKS2)

BLK = 2048


def _dense1_body(x_ref, a0_ref, a1_ref, d0_ref, ws1_ref, wn1_ref,
                 b1_ref, wn2_ref, h1_ref, z2_ref, rdeg_ref):
  a = jnp.concatenate([a0_ref[0], a1_ref[0]], axis=1)
  deg = jnp.maximum(d0_ref[0][:, 0:1], 1.0)
  rdeg = 1.0 / deg
  nbar = a * rdeg
  h1 = x_ref[...] @ ws1_ref[...] + nbar @ wn1_ref[...] + b1_ref[...]
  h1 = jnp.maximum(h1, 0.0)
  h1_ref[...] = h1
  z2_ref[...] = h1 @ wn2_ref[...]
  rdeg_ref[...] = jnp.broadcast_to(rdeg, (BLK, NUM_CLASSES))


def _dense1(x_pad, agg1, deg, w_self1, w_neigh1, b1, w_neigh2):
  grid = (NP // BLK,)
  return pl.pallas_call(
      _dense1_body,
      grid=grid,
      in_specs=[
          pl.BlockSpec((BLK, IN_FEATS), lambda i: (i, 0)),
          pl.BlockSpec((1, BLK, HW), lambda i: (0, i, 0)),
          pl.BlockSpec((1, BLK, HW), lambda i: (1, i, 0)),
          pl.BlockSpec((1, BLK, WD), lambda i: (0, i, 0)),
          pl.BlockSpec((IN_FEATS, H_FEATS), lambda i: (0, 0)),
          pl.BlockSpec((IN_FEATS, H_FEATS), lambda i: (0, 0)),
          pl.BlockSpec((1, H_FEATS), lambda i: (0, 0)),
          pl.BlockSpec((H_FEATS, NUM_CLASSES), lambda i: (0, 0)),
      ],
      out_specs=[
          pl.BlockSpec((BLK, H_FEATS), lambda i: (i, 0)),
          pl.BlockSpec((BLK, NUM_CLASSES), lambda i: (i, 0)),
          pl.BlockSpec((BLK, NUM_CLASSES), lambda i: (i, 0)),
      ],
      out_shape=[
          jax.ShapeDtypeStruct((NP, H_FEATS), jnp.float32),
          jax.ShapeDtypeStruct((NP, NUM_CLASSES), jnp.float32),
          jax.ShapeDtypeStruct((NP, NUM_CLASSES), jnp.float32),
      ],
  )(x_pad, agg1, agg1, deg, w_self1, w_neigh1, b1, w_neigh2)


def _dense2_body(h1_ref, g0_ref, g1_ref, rdeg_ref, ws2_ref, b2_ref, out_ref):
  aggz = (g0_ref[0] + g1_ref[0]) * rdeg_ref[...]
  out_ref[...] = h1_ref[...] @ ws2_ref[...] + aggz + b2_ref[...]


def _dense2(h1, agg2, rdeg, w_self2, b2):
  grid = (NP // BLK,)
  return pl.pallas_call(
      _dense2_body,
      grid=grid,
      in_specs=[
          pl.BlockSpec((BLK, H_FEATS), lambda i: (i, 0)),
          pl.BlockSpec((1, BLK, NUM_CLASSES), lambda i: (0, i, 0)),
          pl.BlockSpec((1, BLK, NUM_CLASSES), lambda i: (1, i, 0)),
          pl.BlockSpec((BLK, NUM_CLASSES), lambda i: (i, 0)),
          pl.BlockSpec((H_FEATS, NUM_CLASSES), lambda i: (0, 0)),
          pl.BlockSpec((1, NUM_CLASSES), lambda i: (0, 0)),
      ],
      out_specs=pl.BlockSpec((BLK, NUM_CLASSES), lambda i: (i, 0)),
      out_shape=jax.ShapeDtypeStruct((NP, NUM_CLASSES), jnp.float32),
  )(h1, agg2, agg2, rdeg, w_self2, b2)


def _pad_edges(src, dst, ep, nsplit, nchunks, bsz):
  pad = ep - N_EDGES
  src_p = jnp.concatenate([src, jnp.zeros((pad,), jnp.int32)])
  dst_p = jnp.concatenate([dst, jnp.full((pad,), NP - 1, jnp.int32)])
  return src_p.reshape(nsplit, nchunks, bsz), dst_p.reshape(nsplit, nchunks, bsz)


@jax.jit
def _run(in_feat, edge_index, w_self1, w_neigh1, b1, w_self2, w_neigh2, b2):
  src = edge_index[0].astype(jnp.int32)
  dst = edge_index[1].astype(jnp.int32)
  pk = src * 16384 + dst
  pk1 = jnp.concatenate(
      [pk, jnp.full((EP1 - N_EDGES,), NP - 1, jnp.int32)]
  ).reshape(NS, CHUNKS1, B1)
  src2, dst2 = _pad_edges(src, dst, EP2, NT, CHUNKS2, B2)
  x_pad = jnp.pad(in_feat, ((0, NP - N_NODES), (0, 0)))
  x_sc = x_pad.reshape(NP, NC, HW).transpose(1, 0, 2)

  a1, dg = _agg1(pk1, x_sc)
  h1, z2, rdeg = _dense1(x_pad, a1, dg, w_self1, w_neigh1,
                         b1.reshape(1, H_FEATS), w_neigh2)
  a2 = _agg2(src2, dst2, z2)
  out = _dense2(h1, a2, rdeg, w_self2, b2.reshape(1, NUM_CLASSES))
  return out[:N_NODES]


def kernel(in_feat, edge_index, W_self1, W_neigh1, b1, W_self2, W_neigh2, b2):
  return _run(in_feat, edge_index, W_self1, W_neigh1, b1, W_self2, W_neigh2,
              b2)


# speedup vs baseline: 10.8617x; 1.0701x over previous
import functools

import jax
import jax.numpy as jnp
from jax import lax
from jax.experimental import pallas as pl
from jax.experimental.pallas import tpu as pltpu
from jax.experimental.pallas import tpu_sc as plsc

N_NODES = 10000
N_EDGES = 320000
IN_FEATS = 128
H_FEATS = 128
NUM_CLASSES = 16

NP = 10240
NC = 2
NS = 16
NT = NC * NS
ROWS_PER_TILE = NP // NS
NSLOT = 4

B1, CHUNKS1 = 64, 316
B2, CHUNKS2 = 128, 80
EP1 = NS * CHUNKS1 * B1
EP2 = NT * CHUNKS2 * B2
WD = 16
HW = IN_FEATS // 2


def _make_sc_agg1():
  mesh = plsc.VectorSubcoreMesh(core_axis_name="c", subcore_axis_name="s")

  @functools.partial(
      pl.kernel,
      out_type=[
          jax.ShapeDtypeStruct((NC, NP, HW), jnp.float32),
          jax.ShapeDtypeStruct((NC, NP, WD), jnp.float32),
      ],
      mesh=mesh,
      compiler_params=pltpu.CompilerParams(use_tc_tiling_on_sc=False),
      scratch_types=[
          pltpu.VMEM((CHUNKS1, B1), jnp.int32),
          [pltpu.VMEM((B1,), jnp.int32) for _ in range(NSLOT)],
          [pltpu.VMEM((B1,), jnp.int32) for _ in range(NSLOT)],
          [pltpu.VMEM((B1, HW), jnp.float32) for _ in range(NSLOT)],
          pltpu.VMEM((B1, WD), jnp.float32),
          pltpu.VMEM_SHARED((NP, HW), jnp.float32),
          pltpu.VMEM_SHARED((NP, HW), jnp.float32),
          pltpu.VMEM_SHARED((NP, WD), jnp.float32),
          [pltpu.SemaphoreType.DMA for _ in range(NSLOT)],
          [pltpu.SemaphoreType.DMA for _ in range(NSLOT)],
          [pltpu.SemaphoreType.DMA for _ in range(NSLOT)],
      ],
  )
  def agg1_body(pk_hbm, x_hbm, out_hbm, deg_hbm, pidx, sidx, didx, rows,
                ones, table, acc, dacc, gsem, ssem, dsem):
    cid = lax.axis_index("c")
    sid = lax.axis_index("s")
    base = sid * ROWS_PER_TILE

    tbl_slab = table.at[pl.ds(base, ROWS_PER_TILE)]
    pltpu.async_copy(x_hbm.at[cid, pl.ds(base, ROWS_PER_TILE)], tbl_slab,
                     gsem[0])
    pltpu.async_copy(pk_hbm.at[sid], pidx, gsem[1])

    def unpack(j, k):
      for c in range(B1 // 16):
        v = pidx[j, pl.ds(c * 16, 16)]
        sidx[k][pl.ds(c * 16, 16)] = lax.shift_right_logical(v, 14)
        didx[k][pl.ds(c * 16, 16)] = lax.bitwise_and(v, 16383)

    zero = jnp.zeros((16,), jnp.float32)
    one = jnp.full((16,), 1.0, jnp.float32)
    for r in range(B1):
      for c in range(HW // 16):
        rows[0][r, pl.ds(c * 16, 16)] = zero
      ones[r, pl.ds(0, WD)] = zero

    nzb = ROWS_PER_TILE // B1
    for i in range(nzb):
      k = i % NSLOT
      if i >= NSLOT:
        pltpu.make_async_copy(
            rows[0], acc.at[pl.ds(base + (i - NSLOT) * B1, B1)],
            ssem[k]).wait()
        pltpu.make_async_copy(
            ones, dacc.at[pl.ds(base + (i - NSLOT) * B1, B1)],
            dsem[k]).wait()
      pltpu.async_copy(rows[0], acc.at[pl.ds(base + i * B1, B1)], ssem[k])
      pltpu.async_copy(ones, dacc.at[pl.ds(base + i * B1, B1)], dsem[k])
    for i in range(nzb - NSLOT, nzb):
      k = i % NSLOT
      pltpu.make_async_copy(rows[0], acc.at[pl.ds(base + i * B1, B1)],
                            ssem[k]).wait()
      pltpu.make_async_copy(ones, dacc.at[pl.ds(base + i * B1, B1)],
                            dsem[k]).wait()
    for r in range(B1):
      ones[r, pl.ds(0, WD)] = one
    pltpu.make_async_copy(x_hbm.at[cid, pl.ds(base, ROWS_PER_TILE)],
                          tbl_slab, gsem[0]).wait()
    pltpu.make_async_copy(pk_hbm.at[sid], pidx, gsem[1]).wait()
    plsc.subcore_barrier()

    def gather_start(k):
      pltpu.async_copy(table.at[sidx[k]], rows[k], gsem[k])

    def gather_wait(k):
      pltpu.make_async_copy(table.at[sidx[k]], rows[k], gsem[k]).wait()

    def scatter_start(k):
      pltpu.async_copy(rows[k], acc.at[didx[k]], ssem[k], add=True)

      @pl.when(cid == (k % 2))
      def _():
        pltpu.async_copy(ones, dacc.at[didx[k]], dsem[k], add=True)

    def scatter_wait(k):
      pltpu.make_async_copy(rows[k], acc.at[didx[k]], ssem[k]).wait()

      @pl.when(cid == (k % 2))
      def _():
        pltpu.make_async_copy(ones, dacc.at[didx[k]], dsem[k]).wait()

    nr = CHUNKS1 // NSLOT
    for k in range(NSLOT):
      unpack(k, k)
      gather_start(k)

    def edge_round(g, _):
      for k in range(NSLOT):
        gather_wait(k)
        scatter_start(k)
      for k in range(NSLOT):
        scatter_wait(k)
        unpack(NSLOT * g + k + NSLOT, k)
        gather_start(k)
      return 0

    lax.fori_loop(0, nr - 1, edge_round, 0)
    for k in range(NSLOT):
      gather_wait(k)
      scatter_start(k)
    for k in range(NSLOT):
      scatter_wait(k)
    plsc.subcore_barrier()

    pltpu.sync_copy(acc.at[pl.ds(base, ROWS_PER_TILE)],
                    out_hbm.at[cid, pl.ds(base, ROWS_PER_TILE)])
    pltpu.sync_copy(dacc.at[pl.ds(base, ROWS_PER_TILE)],
                    deg_hbm.at[cid, pl.ds(base, ROWS_PER_TILE)])

  return agg1_body


def _make_sc_agg2(width, bsz, chunks):
  mesh = plsc.VectorSubcoreMesh(core_axis_name="c", subcore_axis_name="s")

  @functools.partial(
      pl.kernel,
      out_type=jax.ShapeDtypeStruct((NC, NP, width), jnp.float32),
      mesh=mesh,
      compiler_params=pltpu.CompilerParams(use_tc_tiling_on_sc=False),
      scratch_types=[
          pltpu.VMEM((chunks, bsz), jnp.int32),
          pltpu.VMEM((chunks, bsz), jnp.int32),
          [pltpu.VMEM((bsz, width), jnp.float32) for _ in range(NSLOT)],
          pltpu.VMEM_SHARED((NP, width), jnp.float32),
          pltpu.VMEM_SHARED((NP, width), jnp.float32),
          [pltpu.SemaphoreType.DMA for _ in range(NSLOT)],
          [pltpu.SemaphoreType.DMA for _ in range(NSLOT)],
      ],
  )
  def agg_body(src_hbm, dst_hbm, z_hbm, out_hbm, sidx, didx, rows, table,
               acc, gsem, ssem):
    cid = lax.axis_index("c")
    sid = lax.axis_index("s")
    wid = cid * NS + sid
    base = sid * ROWS_PER_TILE

    tbl_slab = table.at[pl.ds(base, ROWS_PER_TILE)]
    pltpu.async_copy(src_hbm.at[wid], sidx, gsem[0])
    pltpu.async_copy(dst_hbm.at[wid], didx, gsem[1])
    pltpu.async_copy(z_hbm.at[pl.ds(base, ROWS_PER_TILE)], tbl_slab, gsem[2])

    zero = jnp.zeros((16,), jnp.float32)
    for r in range(bsz):
      for c in range(width // 16):
        rows[0][r, pl.ds(c * 16, 16)] = zero

    nzb = ROWS_PER_TILE // bsz
    for i in range(nzb):
      pltpu.async_copy(rows[0], acc.at[pl.ds(base + i * bsz, bsz)],
                       ssem[i % NSLOT])
    for i in range(nzb):
      pltpu.make_async_copy(rows[0], acc.at[pl.ds(base + i * bsz, bsz)],
                            ssem[i % NSLOT]).wait()
    pltpu.make_async_copy(src_hbm.at[wid], sidx, gsem[0]).wait()
    pltpu.make_async_copy(dst_hbm.at[wid], didx, gsem[1]).wait()
    pltpu.make_async_copy(z_hbm.at[pl.ds(base, ROWS_PER_TILE)], tbl_slab,
                          gsem[2]).wait()
    plsc.subcore_barrier()

    def gather_start(j, k):
      pltpu.async_copy(table.at[sidx.at[j]], rows[k], gsem[k])

    def gather_wait(j, k):
      pltpu.make_async_copy(table.at[sidx.at[j]], rows[k], gsem[k]).wait()

    def scatter_start(j, k):
      pltpu.async_copy(rows[k], acc.at[didx.at[j]], ssem[k], add=True)

    def scatter_wait(j, k):
      pltpu.make_async_copy(rows[k], acc.at[didx.at[j]], ssem[k]).wait()

    nr = chunks // NSLOT
    for k in range(NSLOT):
      gather_start(k, k)

    def edge_round(g, _):
      for k in range(NSLOT):
        j = NSLOT * g + k
        gather_wait(j, k)
        scatter_start(j, k)
      for k in range(NSLOT):
        j = NSLOT * g + k
        scatter_wait(j, k)
        gather_start(j + NSLOT, k)
      return 0

    lax.fori_loop(0, nr - 1, edge_round, 0)
    for k in range(NSLOT):
      j = NSLOT * (nr - 1) + k
      gather_wait(j, k)
      scatter_start(j, k)
    for k in range(NSLOT):
      j = NSLOT * (nr - 1) + k
      scatter_wait(j, k)
    plsc.subcore_barrier()

    pltpu.sync_copy(acc.at[pl.ds(base, ROWS_PER_TILE)],
                    out_hbm.at[cid, pl.ds(base, ROWS_PER_TILE)])

  return agg_body


_agg1 = _make_sc_agg1()
_agg2 = _make_sc_agg2(NUM_CLASSES, B2, CHUNKS2)

BLK = 2048


def _dense1_body(x_ref, a0_ref, a1_ref, d0_ref, d1_ref, ws1_ref, wn1_ref,
                 b1_ref, wn2_ref, h1_ref, z2_ref, rdeg_ref):
  a = jnp.concatenate([a0_ref[0], a1_ref[0]], axis=1)
  deg = jnp.maximum(d0_ref[0][:, 0:1] + d1_ref[0][:, 0:1], 1.0)
  rdeg = 1.0 / deg
  nbar = a * rdeg
  h1 = x_ref[...] @ ws1_ref[...] + nbar @ wn1_ref[...] + b1_ref[...]
  h1 = jnp.maximum(h1, 0.0)
  h1_ref[...] = h1
  z2_ref[...] = h1 @ wn2_ref[...]
  rdeg_ref[...] = jnp.broadcast_to(rdeg, (BLK, NUM_CLASSES))


def _dense1(x_pad, agg1, deg, w_self1, w_neigh1, b1, w_neigh2):
  grid = (NP // BLK,)
  return pl.pallas_call(
      _dense1_body,
      grid=grid,
      in_specs=[
          pl.BlockSpec((BLK, IN_FEATS), lambda i: (i, 0)),
          pl.BlockSpec((1, BLK, HW), lambda i: (0, i, 0)),
          pl.BlockSpec((1, BLK, HW), lambda i: (1, i, 0)),
          pl.BlockSpec((1, BLK, WD), lambda i: (0, i, 0)),
          pl.BlockSpec((1, BLK, WD), lambda i: (1, i, 0)),
          pl.BlockSpec((IN_FEATS, H_FEATS), lambda i: (0, 0)),
          pl.BlockSpec((IN_FEATS, H_FEATS), lambda i: (0, 0)),
          pl.BlockSpec((1, H_FEATS), lambda i: (0, 0)),
          pl.BlockSpec((H_FEATS, NUM_CLASSES), lambda i: (0, 0)),
      ],
      out_specs=[
          pl.BlockSpec((BLK, H_FEATS), lambda i: (i, 0)),
          pl.BlockSpec((BLK, NUM_CLASSES), lambda i: (i, 0)),
          pl.BlockSpec((BLK, NUM_CLASSES), lambda i: (i, 0)),
      ],
      out_shape=[
          jax.ShapeDtypeStruct((NP, H_FEATS), jnp.float32),
          jax.ShapeDtypeStruct((NP, NUM_CLASSES), jnp.float32),
          jax.ShapeDtypeStruct((NP, NUM_CLASSES), jnp.float32),
      ],
  )(x_pad, agg1, agg1, deg, deg, w_self1, w_neigh1, b1, w_neigh2)


def _dense2_body(h1_ref, g0_ref, g1_ref, rdeg_ref, ws2_ref, b2_ref, out_ref):
  aggz = (g0_ref[0] + g1_ref[0]) * rdeg_ref[...]
  out_ref[...] = h1_ref[...] @ ws2_ref[...] + aggz + b2_ref[...]


def _dense2(h1, agg2, rdeg, w_self2, b2):
  grid = (NP // BLK,)
  return pl.pallas_call(
      _dense2_body,
      grid=grid,
      in_specs=[
          pl.BlockSpec((BLK, H_FEATS), lambda i: (i, 0)),
          pl.BlockSpec((1, BLK, NUM_CLASSES), lambda i: (0, i, 0)),
          pl.BlockSpec((1, BLK, NUM_CLASSES), lambda i: (1, i, 0)),
          pl.BlockSpec((BLK, NUM_CLASSES), lambda i: (i, 0)),
          pl.BlockSpec((H_FEATS, NUM_CLASSES), lambda i: (0, 0)),
          pl.BlockSpec((1, NUM_CLASSES), lambda i: (0, 0)),
      ],
      out_specs=pl.BlockSpec((BLK, NUM_CLASSES), lambda i: (i, 0)),
      out_shape=jax.ShapeDtypeStruct((NP, NUM_CLASSES), jnp.float32),
  )(h1, agg2, agg2, rdeg, w_self2, b2)


def _pad_edges(src, dst, ep, nsplit, nchunks, bsz):
  pad = ep - N_EDGES
  src_p = jnp.concatenate([src, jnp.zeros((pad,), jnp.int32)])
  dst_p = jnp.concatenate([dst, jnp.full((pad,), NP - 1, jnp.int32)])
  return src_p.reshape(nsplit, nchunks, bsz), dst_p.reshape(nsplit, nchunks, bsz)


@jax.jit
def _run(in_feat, edge_index, w_self1, w_neigh1, b1, w_self2, w_neigh2, b2):
  src = edge_index[0].astype(jnp.int32)
  dst = edge_index[1].astype(jnp.int32)
  pk = src * 16384 + dst
  pk1 = jnp.concatenate(
      [pk, jnp.full((EP1 - N_EDGES,), NP - 1, jnp.int32)]
  ).reshape(NS, CHUNKS1, B1)
  src2, dst2 = _pad_edges(src, dst, EP2, NT, CHUNKS2, B2)
  x_pad = jnp.pad(in_feat, ((0, NP - N_NODES), (0, 0)))
  x_sc = x_pad.reshape(NP, NC, HW).transpose(1, 0, 2)

  a1, dg = _agg1(pk1, x_sc)
  h1, z2, rdeg = _dense1(x_pad, a1, dg, w_self1, w_neigh1,
                         b1.reshape(1, H_FEATS), w_neigh2)
  a2 = _agg2(src2, dst2, z2)
  out = _dense2(h1, a2, rdeg, w_self2, b2.reshape(1, NUM_CLASSES))
  return out[:N_NODES]


def kernel(in_feat, edge_index, W_self1, W_neigh1, b1, W_self2, W_neigh2, b2):
  return _run(in_feat, edge_index, W_self1, W_neigh1, b1, W_self2, W_neigh2,
              b2)


# speedup vs baseline: 10.8909x; 1.0027x over previous
import functools

import jax
import jax.numpy as jnp
from jax import lax
from jax.experimental import pallas as pl
from jax.experimental.pallas import tpu as pltpu
from jax.experimental.pallas import tpu_sc as plsc

N_NODES = 10000
N_EDGES = 320000
IN_FEATS = 128
H_FEATS = 128
NUM_CLASSES = 16

NP = 10240
NC = 2
NS = 16
NT = NC * NS
ROWS_PER_TILE = NP // NS
NSLOT = 4

B1, CHUNKS1 = 64, 316
B2, CHUNKS2 = 128, 80
EP1 = NS * CHUNKS1 * B1
EP2 = NT * CHUNKS2 * B2
WD = 16
HW = IN_FEATS // 2


def _make_sc_agg1():
  mesh = plsc.VectorSubcoreMesh(core_axis_name="c", subcore_axis_name="s")

  @functools.partial(
      pl.kernel,
      out_type=[
          jax.ShapeDtypeStruct((NC, NP, HW), jnp.float32),
          jax.ShapeDtypeStruct((NC, NP, WD), jnp.float32),
      ],
      mesh=mesh,
      compiler_params=pltpu.CompilerParams(use_tc_tiling_on_sc=False),
      scratch_types=[
          pltpu.VMEM((CHUNKS1, B1), jnp.int32),
          [pltpu.VMEM((B1,), jnp.int32) for _ in range(NSLOT)],
          [pltpu.VMEM((B1,), jnp.int32) for _ in range(NSLOT)],
          [pltpu.VMEM((B1, HW), jnp.float32) for _ in range(NSLOT)],
          pltpu.VMEM((B1, WD), jnp.float32),
          pltpu.VMEM_SHARED((NP, HW), jnp.float32),
          pltpu.VMEM_SHARED((NP, HW), jnp.float32),
          pltpu.VMEM_SHARED((NP, WD), jnp.float32),
          [pltpu.SemaphoreType.DMA for _ in range(NSLOT)],
          [pltpu.SemaphoreType.DMA for _ in range(NSLOT)],
          [pltpu.SemaphoreType.DMA for _ in range(NSLOT)],
      ],
  )
  def agg1_body(pk_hbm, x_hbm, out_hbm, deg_hbm, pidx, sidx, didx, rows,
                ones, table, acc, dacc, gsem, ssem, dsem):
    cid = lax.axis_index("c")
    sid = lax.axis_index("s")
    base = sid * ROWS_PER_TILE

    tbl_slab = table.at[pl.ds(base, ROWS_PER_TILE)]
    pltpu.async_copy(x_hbm.at[cid, pl.ds(base, ROWS_PER_TILE)], tbl_slab,
                     gsem[0])
    pltpu.async_copy(pk_hbm.at[sid], pidx, gsem[1])

    def unpack(j, k):
      for c in range(B1 // 16):
        v = pidx[j, pl.ds(c * 16, 16)]
        sidx[k][pl.ds(c * 16, 16)] = lax.shift_right_logical(v, 14)
        didx[k][pl.ds(c * 16, 16)] = lax.bitwise_and(v, 16383)

    zero = jnp.zeros((16,), jnp.float32)
    one = jnp.full((16,), 1.0, jnp.float32)
    for r in range(B1):
      for c in range(HW // 16):
        rows[0][r, pl.ds(c * 16, 16)] = zero
      ones[r, pl.ds(0, WD)] = zero

    nzb = ROWS_PER_TILE // B1
    for i in range(nzb):
      k = i % NSLOT
      if i >= NSLOT:
        pltpu.make_async_copy(
            rows[0], acc.at[pl.ds(base + (i - NSLOT) * B1, B1)],
            ssem[k]).wait()
        pltpu.make_async_copy(
            ones, dacc.at[pl.ds(base + (i - NSLOT) * B1, B1)],
            dsem[k]).wait()
      pltpu.async_copy(rows[0], acc.at[pl.ds(base + i * B1, B1)], ssem[k])
      pltpu.async_copy(ones, dacc.at[pl.ds(base + i * B1, B1)], dsem[k])
    for i in range(nzb - NSLOT, nzb):
      k = i % NSLOT
      pltpu.make_async_copy(rows[0], acc.at[pl.ds(base + i * B1, B1)],
                            ssem[k]).wait()
      pltpu.make_async_copy(ones, dacc.at[pl.ds(base + i * B1, B1)],
                            dsem[k]).wait()
    for r in range(B1):
      ones[r, pl.ds(0, WD)] = one
    pltpu.make_async_copy(x_hbm.at[cid, pl.ds(base, ROWS_PER_TILE)],
                          tbl_slab, gsem[0]).wait()
    pltpu.make_async_copy(pk_hbm.at[sid], pidx, gsem[1]).wait()
    plsc.subcore_barrier()

    def gather_start(k):
      pltpu.async_copy(table.at[sidx[k]], rows[k], gsem[k])

    def gather_wait(k):
      pltpu.make_async_copy(table.at[sidx[k]], rows[k], gsem[k]).wait()

    def scatter_start(k):
      pltpu.async_copy(rows[k], acc.at[didx[k]], ssem[k], add=True)

      @pl.when(cid == (k % 2))
      def _():
        pltpu.async_copy(ones, dacc.at[didx[k]], dsem[k], add=True)

    def scatter_wait(k):
      pltpu.make_async_copy(rows[k], acc.at[didx[k]], ssem[k]).wait()

      @pl.when(cid == (k % 2))
      def _():
        pltpu.make_async_copy(ones, dacc.at[didx[k]], dsem[k]).wait()

    nr = CHUNKS1 // NSLOT
    for k in range(NSLOT):
      unpack(k, k)
      gather_start(k)

    def edge_round(g, _):
      for k in range(NSLOT):
        gather_wait(k)
        scatter_start(k)
      for k in range(NSLOT):
        scatter_wait(k)
        unpack(NSLOT * g + k + NSLOT, k)
        gather_start(k)
      return 0

    lax.fori_loop(0, nr - 1, edge_round, 0)
    for k in range(NSLOT):
      gather_wait(k)
      scatter_start(k)
    for k in range(NSLOT):
      scatter_wait(k)
    plsc.subcore_barrier()

    pltpu.sync_copy(acc.at[pl.ds(base, ROWS_PER_TILE)],
                    out_hbm.at[cid, pl.ds(base, ROWS_PER_TILE)])
    pltpu.sync_copy(dacc.at[pl.ds(base, ROWS_PER_TILE)],
                    deg_hbm.at[cid, pl.ds(base, ROWS_PER_TILE)])

  return agg1_body


def _make_sc_agg2(width, bsz, chunks):
  mesh = plsc.VectorSubcoreMesh(core_axis_name="c", subcore_axis_name="s")

  @functools.partial(
      pl.kernel,
      out_type=jax.ShapeDtypeStruct((NC, NP, width), jnp.float32),
      mesh=mesh,
      compiler_params=pltpu.CompilerParams(use_tc_tiling_on_sc=False),
      scratch_types=[
          pltpu.VMEM((chunks, bsz), jnp.int32),
          pltpu.VMEM((chunks, bsz), jnp.int32),
          [pltpu.VMEM((bsz, width), jnp.float32) for _ in range(NSLOT)],
          pltpu.VMEM_SHARED((NP, width), jnp.float32),
          pltpu.VMEM_SHARED((NP, width), jnp.float32),
          [pltpu.SemaphoreType.DMA for _ in range(NSLOT)],
          [pltpu.SemaphoreType.DMA for _ in range(NSLOT)],
      ],
  )
  def agg_body(src_hbm, dst_hbm, z_hbm, out_hbm, sidx, didx, rows, table,
               acc, gsem, ssem):
    cid = lax.axis_index("c")
    sid = lax.axis_index("s")
    wid = cid * NS + sid
    base = sid * ROWS_PER_TILE

    tbl_slab = table.at[pl.ds(base, ROWS_PER_TILE)]
    pltpu.async_copy(src_hbm.at[wid], sidx, gsem[0])
    pltpu.async_copy(dst_hbm.at[wid], didx, gsem[1])
    pltpu.async_copy(z_hbm.at[pl.ds(base, ROWS_PER_TILE)], tbl_slab, gsem[2])

    zero = jnp.zeros((16,), jnp.float32)
    for r in range(bsz):
      for c in range(width // 16):
        rows[0][r, pl.ds(c * 16, 16)] = zero

    nzb = ROWS_PER_TILE // bsz
    for i in range(nzb):
      pltpu.async_copy(rows[0], acc.at[pl.ds(base + i * bsz, bsz)],
                       ssem[i % NSLOT])
    for i in range(nzb):
      pltpu.make_async_copy(rows[0], acc.at[pl.ds(base + i * bsz, bsz)],
                            ssem[i % NSLOT]).wait()
    pltpu.make_async_copy(src_hbm.at[wid], sidx, gsem[0]).wait()
    pltpu.make_async_copy(dst_hbm.at[wid], didx, gsem[1]).wait()
    pltpu.make_async_copy(z_hbm.at[pl.ds(base, ROWS_PER_TILE)], tbl_slab,
                          gsem[2]).wait()
    plsc.subcore_barrier()

    def gather_start(j, k):
      pltpu.async_copy(table.at[sidx.at[j]], rows[k], gsem[k])

    def gather_wait(j, k):
      pltpu.make_async_copy(table.at[sidx.at[j]], rows[k], gsem[k]).wait()

    def scatter_start(j, k):
      pltpu.async_copy(rows[k], acc.at[didx.at[j]], ssem[k], add=True)

    def scatter_wait(j, k):
      pltpu.make_async_copy(rows[k], acc.at[didx.at[j]], ssem[k]).wait()

    nr = chunks // NSLOT
    for k in range(NSLOT):
      gather_start(k, k)

    def edge_round(g, _):
      for k in range(NSLOT):
        j = NSLOT * g + k
        gather_wait(j, k)
        scatter_start(j, k)
      for k in range(NSLOT):
        j = NSLOT * g + k
        scatter_wait(j, k)
        gather_start(j + NSLOT, k)
      return 0

    lax.fori_loop(0, nr - 1, edge_round, 0)
    for k in range(NSLOT):
      j = NSLOT * (nr - 1) + k
      gather_wait(j, k)
      scatter_start(j, k)
    for k in range(NSLOT):
      j = NSLOT * (nr - 1) + k
      scatter_wait(j, k)
    plsc.subcore_barrier()

    pltpu.sync_copy(acc.at[pl.ds(base, ROWS_PER_TILE)],
                    out_hbm.at[cid, pl.ds(base, ROWS_PER_TILE)])

  return agg_body


_agg1 = _make_sc_agg1()
_agg2 = _make_sc_agg2(NUM_CLASSES, B2, CHUNKS2)

BLK = 2048


def _dense1_body(x_ref, a0_ref, a1_ref, d0_ref, d1_ref, ws1_ref, wn1_ref,
                 b1_ref, wn2_ref, ws2_ref, b2_ref, z2_ref, p2_ref, rdeg_ref):
  a = jnp.concatenate([a0_ref[0], a1_ref[0]], axis=1)
  deg = jnp.maximum(d0_ref[0][:, 0:1] + d1_ref[0][:, 0:1], 1.0)
  rdeg = 1.0 / deg
  nbar = a * rdeg
  h1 = x_ref[...] @ ws1_ref[...] + nbar @ wn1_ref[...] + b1_ref[...]
  h1 = jnp.maximum(h1, 0.0)
  z2_ref[...] = h1 @ wn2_ref[...]
  p2_ref[...] = h1 @ ws2_ref[...] + b2_ref[...]
  rdeg_ref[...] = jnp.broadcast_to(rdeg, (BLK, NUM_CLASSES))


def _dense1(x_pad, agg1, deg, w_self1, w_neigh1, b1, w_neigh2, w_self2, b2):
  grid = (NP // BLK,)
  return pl.pallas_call(
      _dense1_body,
      grid=grid,
      in_specs=[
          pl.BlockSpec((BLK, IN_FEATS), lambda i: (i, 0)),
          pl.BlockSpec((1, BLK, HW), lambda i: (0, i, 0)),
          pl.BlockSpec((1, BLK, HW), lambda i: (1, i, 0)),
          pl.BlockSpec((1, BLK, WD), lambda i: (0, i, 0)),
          pl.BlockSpec((1, BLK, WD), lambda i: (1, i, 0)),
          pl.BlockSpec((IN_FEATS, H_FEATS), lambda i: (0, 0)),
          pl.BlockSpec((IN_FEATS, H_FEATS), lambda i: (0, 0)),
          pl.BlockSpec((1, H_FEATS), lambda i: (0, 0)),
          pl.BlockSpec((H_FEATS, NUM_CLASSES), lambda i: (0, 0)),
          pl.BlockSpec((H_FEATS, NUM_CLASSES), lambda i: (0, 0)),
          pl.BlockSpec((1, NUM_CLASSES), lambda i: (0, 0)),
      ],
      out_specs=[
          pl.BlockSpec((BLK, NUM_CLASSES), lambda i: (i, 0)),
          pl.BlockSpec((BLK, NUM_CLASSES), lambda i: (i, 0)),
          pl.BlockSpec((BLK, NUM_CLASSES), lambda i: (i, 0)),
      ],
      out_shape=[
          jax.ShapeDtypeStruct((NP, NUM_CLASSES), jnp.float32),
          jax.ShapeDtypeStruct((NP, NUM_CLASSES), jnp.float32),
          jax.ShapeDtypeStruct((NP, NUM_CLASSES), jnp.float32),
      ],
  )(x_pad, agg1, agg1, deg, deg, w_self1, w_neigh1, b1, w_neigh2, w_self2,
    b2)


BLK2 = 2000


def _dense2_body(p2_ref, g0_ref, g1_ref, rdeg_ref, out_ref):
  out_ref[...] = p2_ref[...] + (g0_ref[0] + g1_ref[0]) * rdeg_ref[...]


def _dense2(p2, agg2, rdeg):
  grid = (N_NODES // BLK2,)
  return pl.pallas_call(
      _dense2_body,
      grid=grid,
      in_specs=[
          pl.BlockSpec((BLK2, NUM_CLASSES), lambda i: (i, 0)),
          pl.BlockSpec((1, BLK2, NUM_CLASSES), lambda i: (0, i, 0)),
          pl.BlockSpec((1, BLK2, NUM_CLASSES), lambda i: (1, i, 0)),
          pl.BlockSpec((BLK2, NUM_CLASSES), lambda i: (i, 0)),
      ],
      out_specs=pl.BlockSpec((BLK2, NUM_CLASSES), lambda i: (i, 0)),
      out_shape=jax.ShapeDtypeStruct((N_NODES, NUM_CLASSES), jnp.float32),
  )(p2, agg2, agg2, rdeg)


def _pad_edges(src, dst, ep, nsplit, nchunks, bsz):
  pad = ep - N_EDGES
  src_p = jnp.concatenate([src, jnp.zeros((pad,), jnp.int32)])
  dst_p = jnp.concatenate([dst, jnp.full((pad,), NP - 1, jnp.int32)])
  return src_p.reshape(nsplit, nchunks, bsz), dst_p.reshape(nsplit, nchunks, bsz)


@jax.jit
def _run(in_feat, edge_index, w_self1, w_neigh1, b1, w_self2, w_neigh2, b2):
  src = edge_index[0].astype(jnp.int32)
  dst = edge_index[1].astype(jnp.int32)
  pk = src * 16384 + dst
  pk1 = jnp.concatenate(
      [pk, jnp.full((EP1 - N_EDGES,), NP - 1, jnp.int32)]
  ).reshape(NS, CHUNKS1, B1)
  src2, dst2 = _pad_edges(src, dst, EP2, NT, CHUNKS2, B2)
  x_pad = jnp.pad(in_feat, ((0, NP - N_NODES), (0, 0)))
  x_sc = x_pad.reshape(NP, NC, HW).transpose(1, 0, 2)

  a1, dg = _agg1(pk1, x_sc)
  z2, p2, rdeg = _dense1(x_pad, a1, dg, w_self1, w_neigh1,
                         b1.reshape(1, H_FEATS), w_neigh2, w_self2,
                         b2.reshape(1, NUM_CLASSES))
  a2 = _agg2(src2, dst2, z2)
  return _dense2(p2, a2, rdeg)


def kernel(in_feat, edge_index, W_self1, W_neigh1, b1, W_self2, W_neigh2, b2):
  return _run(in_feat, edge_index, W_self1, W_neigh1, b1, W_self2, W_neigh2,
              b2)
